# baseline (speedup 1.0000x reference)
import functools

import jax
import jax.numpy as jnp
from jax import lax
from jax.experimental import pallas as pl
from jax.experimental.pallas import tpu as pltpu
from jax.experimental.pallas import tpu_sc as plsc

_NC = 2
_NS = 16
_NW = _NC * _NS
_L = 16

_BB = 128
_FPAD = 32
_PACK = 4


@functools.lru_cache(maxsize=None)
def _build(batch, fields, dim, rows):
    nblocks = batch // _BB
    bpw = nblocks // _NW
    nstream = bpw * fields
    assert nstream % 2 == 0
    nj8 = fields * dim // 8
    mesh = plsc.VectorSubcoreMesh(core_axis_name="c", subcore_axis_name="s")

    @functools.partial(
        pl.kernel,
        out_type=jax.ShapeDtypeStruct((nj8, nblocks, 8, _BB), jnp.float32),
        mesh=mesh,
        scratch_types=[
            pltpu.VMEM((bpw, _FPAD, _BB), jnp.int32),
            pltpu.VMEM((bpw, _FPAD, _BB), jnp.int32),
            pltpu.VMEM((_FPAD, _L), jnp.int32),
            pltpu.VMEM((2, _BB, _BB), jnp.float32),
            pltpu.VMEM((2, dim // 8, 1, 1, 8, _BB), jnp.float32),
            pltpu.SemaphoreType.DMA,
            pltpu.SemaphoreType.DMA,
            pltpu.SemaphoreType.DMA,
            pltpu.SemaphoreType.DMA,
            pltpu.SemaphoreType.DMA,
        ],
        compiler_params=pltpu.CompilerParams(
            use_tc_tiling_on_sc=True, needs_layout_passes=False),
    )
    def gather_kernel(idx_hbm, off_hbm, t4_hbm, out_hbm,
                      slab_s, slab_r, off_v, gbuf, tbuf,
                      sem_in, sem_g0, sem_g1, sem_t0, sem_t1):
        sem_g = (sem_g0, sem_g1)
        sem_t = (sem_t0, sem_t1)
        wid = lax.axis_index("s") * _NC + lax.axis_index("c")
        blk0 = wid * bpw
        lanes = lax.broadcasted_iota(jnp.int32, (_L,), 0)

        pltpu.sync_copy(off_hbm, off_v)
        pltpu.async_copy(idx_hbm.at[pl.ds(pl.multiple_of(blk0, bpw), bpw)],
                         slab_s, sem_in).wait()

        def shift(i, carry):
            blk = i // fields
            f = lax.rem(i, fields)
            off16 = off_v[f, pl.ds(0, _L)]
            for k in range(_BB // _L):
                s = pl.ds(k * _L, _L)
                v = slab_s[blk, f, s] + off16
                slab_s[blk, f, s] = v
                slab_r[blk, f, s] = lax.shift_right_logical(v, 2)
            return carry
        lax.fori_loop(0, nstream, shift, 0)

        def fire(i, q):
            blk = i // fields
            f = lax.rem(i, fields)
            pltpu.async_copy(t4_hbm.at[slab_r.at[blk, f]], gbuf.at[q],
                             sem_g[q])

        def wait_g(q):
            pltpu.make_async_copy(t4_hbm.at[slab_r.at[0, 0]], gbuf.at[q],
                                  sem_g[q]).wait()

        def wait_tiles(q):
            for t in range(dim // 8):
                pltpu.make_async_copy(
                    tbuf.at[q, t], out_hbm.at[pl.ds(0, 1), pl.ds(0, 1)],
                    sem_t[q]).wait()

        def process(i, q):
            blk = i // fields
            f = lax.rem(i, fields)
            for k in range(_BB // _L):
                idx16 = slab_s[blk, f, pl.ds(k * _L, _L)]
                m16 = (idx16 & (_PACK - 1)) * dim
                rowk = lanes + (k * _L)
                for d in range(dim):
                    x = plsc.load_gather(gbuf.at[q], [rowk, m16 + d])
                    tbuf[q, d // 8, 0, 0, d % 8, pl.ds(k * _L, _L)] = x
            bbg = blk0 + blk
            for t in range(dim // 8):
                j8 = f * (dim // 8) + t
                pltpu.async_copy(
                    tbuf.at[q, t],
                    out_hbm.at[pl.ds(j8, 1), pl.ds(bbg, 1)], sem_t[q])

        fire(0, 0)

        def body(u, carry):
            i0 = 2 * u
            fire(i0 + 1, 1)
            wait_g(0)
            @pl.when(u >= 1)
            def _():
                wait_tiles(0)
            process(i0, 0)

            @pl.when(u < (nstream // 2) - 1)
            def _():
                fire(i0 + 2, 0)
            wait_g(1)
            @pl.when(u >= 1)
            def _():
                wait_tiles(1)
            process(i0 + 1, 1)
            return carry

        lax.fori_loop(0, nstream // 2, body, 0)
        wait_tiles(0)
        wait_tiles(1)

    return gather_kernel


def kernel(input, num_classes, table):
    batch, fields = input.shape
    rows, dim = table.shape
    offsets = jnp.concatenate([
        jnp.zeros((1,), dtype=num_classes.dtype),
        jnp.cumsum(num_classes)[:-1],
    ]).astype(jnp.int32)
    off_bc = jnp.zeros((_FPAD, _L), jnp.int32).at[:fields, :].set(
        jnp.broadcast_to(offsets[:, None], (fields, _L)))
    idx_fm = jnp.pad(
        input.T.reshape(fields, batch // _BB, _BB).transpose(1, 0, 2),
        ((0, 0), (0, _FPAD - fields), (0, 0)))
    t4 = table.reshape(rows // _PACK, _PACK * dim)
    out4 = _build(batch, fields, dim, rows)(idx_fm, off_bc, t4)
    return out4.transpose(1, 3, 0, 2).reshape(batch, fields * dim)
